# baseline (speedup 1.0000x reference)
import functools

import jax
import jax.numpy as jnp
from jax import lax
from jax.experimental import pallas as pl
from jax.experimental.pallas import tpu as pltpu
from jax.experimental.pallas import tpu_sc as plsc

_NC = 2
_NS = 16
_NW = _NC * _NS
_L = 16
_CH = 128

_F1 = (1.0, -1.0 / 6, 1.0 / 120, -1.0 / 5040, 1.0 / 362880, -1.0 / 39916800)
_F2 = (0.5, -1.0 / 24, 1.0 / 720, -1.0 / 40320, 1.0 / 3628800,
       -1.0 / 479001600)


def _poly(coeffs, x):
    acc = jnp.full((_L,), coeffs[-1], jnp.float32)
    for c in coeffs[-2::-1]:
        acc = acc * x + jnp.float32(c)
    return acc


def _iota():
    return jnp.arange(_L, dtype=jnp.int32)


@functools.partial(jax.jit, static_argnums=(2,))
def _gather_expmap(indices, table3, bpw):
    b = indices.shape[0]
    nbt = b // 128
    cpw = bpw // _CH
    mesh = plsc.VectorSubcoreMesh(
        core_axis_name="c", subcore_axis_name="s",
        num_cores=_NC, num_subcores=_NS)

    @functools.partial(
        pl.kernel,
        mesh=mesh,
        out_type=jax.ShapeDtypeStruct((3, nbt, 4, 128), jnp.float32),
        scratch_types=[
            pltpu.VMEM((bpw,), jnp.int32),
            pltpu.VMEM((1, bpw), jnp.int32),
            pltpu.VMEM((6, bpw, 8), jnp.float32),
            pltpu.VMEM((3, cpw, 4, 128), jnp.float32),
            pltpu.SemaphoreType.DMA,
        ],
        compiler_params=pltpu.CompilerParams(
            needs_layout_passes=False, use_tc_tiling_on_sc=False),
    )
    def k(idx_hbm, table_hbm, out_hbm, idx_v, units_v, rows_v, out_v, sem):
        wid = lax.axis_index("s") * _NC + lax.axis_index("c")
        pltpu.sync_copy(idx_hbm.at[pl.ds(wid * bpw, bpw)], idx_v)
        for s in range(bpw // _L):
            idx16 = idx_v[pl.ds(s * _L, _L)]
            units_v[0, pl.ds(s * _L, _L)] = idx16 >> 3
        copies = [
            pltpu.async_copy(
                table_hbm.at[c].at[units_v.at[0]],
                rows_v.at[c], sem)
            for c in range(6)
        ]
        for c in range(6):
            copies[c].wait()
        for j in range(cpw):
            for s in range(_CH // _L):
                idx16 = idx_v[pl.ds(j * _CH + s * _L, _L)]
                off = idx16 & 7
                row = _iota() + (j * _CH + s * _L)

                def col(kk):
                    return plsc.load_gather(
                        rows_v,
                        [jnp.full((_L,), kk, jnp.int32), row, off])

                tx, ty, tz = col(0), col(1), col(2)
                wx, wy, wz = col(3), col(4), col(5)
                nr = wx * wx + wy * wy + wz * wz
                x = jnp.maximum(nr, jnp.float32(1e-4))
                f1 = _poly(_F1, x)
                f2 = _poly(_F2, x)
                diag = 1.0 - f2 * nr
                cwx = f2 * wx
                cxy = cwx * wy
                cxz = cwx * wz
                cyz = f2 * wy * wz
                sx = f1 * wx
                sy = f1 * wy
                sz = f1 * wz
                cols = (
                    diag + cwx * wx, cxy - sz, cxz + sy, tx,
                    cxy + sz, diag + f2 * wy * wy, cyz - sx, ty,
                    cxz - sy, cyz + sx, diag + f2 * wz * wz, tz,
                )
                for kk, v in enumerate(cols):
                    out_v[kk // 4, j, kk % 4, pl.ds(s * _L, _L)] = v
        outs = [
            pltpu.async_copy(
                out_v.at[r], out_hbm.at[r, pl.ds(wid * cpw, cpw)], sem)
            for r in range(3)
        ]
        for o in outs:
            o.wait()

    return k(indices, table3)


def kernel(indices, pose_adjustment):
    b = indices.shape[0]
    v, d = pose_adjustment.shape
    table3 = pose_adjustment.T.reshape(d, v // 8, 8)
    out4 = _gather_expmap(indices, table3, b // _NW)
    return lax.reshape(out4, (b, 3, 4), dimensions=(1, 3, 0, 2))


# speedup vs baseline: 1.0146x; 1.0146x over previous
import functools

import jax
import jax.numpy as jnp
from jax import lax
from jax.experimental import pallas as pl
from jax.experimental.pallas import tpu as pltpu
from jax.experimental.pallas import tpu_sc as plsc

_NC = 2
_NS = 16
_NW = _NC * _NS
_L = 16
_CH = 128

_F1 = (1.0, -1.0 / 6, 1.0 / 120, -1.0 / 5040, 1.0 / 362880, -1.0 / 39916800)
_F2 = (0.5, -1.0 / 24, 1.0 / 720, -1.0 / 40320, 1.0 / 3628800,
       -1.0 / 479001600)


def _poly(coeffs, x):
    acc = jnp.full((_L,), coeffs[-1], jnp.float32)
    for c in coeffs[-2::-1]:
        acc = acc * x + jnp.float32(c)
    return acc


def _iota():
    return jnp.arange(_L, dtype=jnp.int32)


@functools.partial(jax.jit, static_argnums=(2,))
def _gather_expmap(indices, table3, bpw):
    b = indices.shape[0]
    nbt = b // 128
    cpw = bpw // _CH
    mesh = plsc.VectorSubcoreMesh(
        core_axis_name="c", subcore_axis_name="s",
        num_cores=_NC, num_subcores=_NS)

    @functools.partial(
        pl.kernel,
        mesh=mesh,
        out_type=jax.ShapeDtypeStruct((3, nbt, 4, 128), jnp.float32),
        scratch_types=[
            pltpu.VMEM((bpw,), jnp.int32),
            pltpu.VMEM((cpw, _CH), jnp.int32),
            pltpu.VMEM((cpw * 6, _CH, 8), jnp.float32),
            pltpu.VMEM((3, cpw, 4, 128), jnp.float32),
            pltpu.SemaphoreType.DMA,
        ],
        compiler_params=pltpu.CompilerParams(
            needs_layout_passes=False, use_tc_tiling_on_sc=False),
    )
    def k(idx_hbm, table_hbm, out_hbm, idx_v, units_v, rows_v, out_v, sem):
        wid = lax.axis_index("s") * _NC + lax.axis_index("c")
        pltpu.sync_copy(idx_hbm.at[pl.ds(wid * bpw, bpw)], idx_v)
        copies = []
        for j in range(cpw):
            for s in range(_CH // _L):
                idx16 = idx_v[pl.ds(j * _CH + s * _L, _L)]
                units_v[j, pl.ds(s * _L, _L)] = idx16 >> 3
            for c in range(6):
                copies.append(pltpu.async_copy(
                    table_hbm.at[c].at[units_v.at[j]],
                    rows_v.at[j * 6 + c], sem))
        for j in range(cpw):
            for c in range(6):
                copies[j * 6 + c].wait()
            for s in range(_CH // _L):
                idx16 = idx_v[pl.ds(j * _CH + s * _L, _L)]
                off = idx16 & 7
                row = _iota() + (s * _L)

                def col(kk):
                    return plsc.load_gather(
                        rows_v,
                        [jnp.full((_L,), j * 6 + kk, jnp.int32), row, off])

                tx, ty, tz = col(0), col(1), col(2)
                wx, wy, wz = col(3), col(4), col(5)
                nr = wx * wx + wy * wy + wz * wz
                x = jnp.maximum(nr, jnp.float32(1e-4))
                f1 = _poly(_F1, x)
                f2 = _poly(_F2, x)
                diag = 1.0 - f2 * nr
                cwx = f2 * wx
                cxy = cwx * wy
                cxz = cwx * wz
                cyz = f2 * wy * wz
                sx = f1 * wx
                sy = f1 * wy
                sz = f1 * wz
                cols = (
                    diag + cwx * wx, cxy - sz, cxz + sy, tx,
                    cxy + sz, diag + f2 * wy * wy, cyz - sx, ty,
                    cxz - sy, cyz + sx, diag + f2 * wz * wz, tz,
                )
                for kk, v in enumerate(cols):
                    out_v[kk // 4, j, kk % 4, pl.ds(s * _L, _L)] = v
        outs = [
            pltpu.async_copy(
                out_v.at[r], out_hbm.at[r, pl.ds(wid * cpw, cpw)], sem)
            for r in range(3)
        ]
        for o in outs:
            o.wait()

    return k(indices, table3)


def kernel(indices, pose_adjustment):
    b = indices.shape[0]
    v, d = pose_adjustment.shape
    table3 = pose_adjustment.T.reshape(d, v // 8, 8)
    out4 = _gather_expmap(indices, table3, b // _NW)
    return lax.reshape(out4, (b, 3, 4), dimensions=(1, 3, 0, 2))
